# baseline (speedup 1.0000x reference)
import functools

import jax
import jax.numpy as jnp
from jax import lax
from jax.experimental import pallas as pl
from jax.experimental.pallas import tpu as pltpu
from jax.experimental.pallas import tpu_sc as plsc

VOCAB = 100000
EMB_DIM = 128
NUM_CLASSES = 64


_ROWS_PER_BLOCK = 2000


def _project_body(emb_ref, w_ref, b_ref, out_ref):
    out_ref[...] = (
        jnp.dot(emb_ref[...], w_ref[...], preferred_element_type=jnp.float32)
        + b_ref[...]
    )


def _project(emb2, w_blk, b2):
    n_blocks = (VOCAB // 2) // _ROWS_PER_BLOCK
    return pl.pallas_call(
        _project_body,
        grid=(n_blocks,),
        in_specs=[
            pl.BlockSpec((_ROWS_PER_BLOCK, 2 * EMB_DIM), lambda i: (i, 0)),
            pl.BlockSpec((2 * EMB_DIM, 2 * NUM_CLASSES), lambda i: (0, 0)),
            pl.BlockSpec((1, 2 * NUM_CLASSES), lambda i: (0, 0)),
        ],
        out_specs=pl.BlockSpec(
            (_ROWS_PER_BLOCK, 2 * NUM_CLASSES), lambda i: (i, 0)
        ),
        out_shape=jax.ShapeDtypeStruct(
            (VOCAB // 2, 2 * NUM_CLASSES), jnp.float32
        ),
    )(emb2, w_blk, b2)


_CB = 128


def _make_gather(B, L, nw):
    n_chunks_total = (B // _CB) * L
    cpw = n_chunks_total // nw
    assert cpw % 2 == 0
    nbc = B // _CB
    mesh = plsc.VectorSubcoreMesh(core_axis_name="c", subcore_axis_name="s")
    nc = mesh.num_cores

    @functools.partial(
        pl.kernel,
        mesh=mesh,
        out_type=jax.ShapeDtypeStruct((L, NUM_CLASSES, B), jnp.float32),
        scratch_types=[
            pltpu.VMEM((cpw, _CB), jnp.int32),
            pltpu.VMEM((_CB,), jnp.int32),
            pltpu.VMEM((_CB,), jnp.int32),
            pltpu.VMEM((_CB, 2 * NUM_CLASSES), jnp.float32),
            pltpu.VMEM((_CB, 2 * NUM_CLASSES), jnp.float32),
            pltpu.VMEM((_CB, 2 * NUM_CLASSES + 16), jnp.float32),
            pltpu.VMEM((_CB, 2 * NUM_CLASSES + 16), jnp.float32),
            pltpu.VMEM((NUM_CLASSES, _CB), jnp.float32),
            pltpu.VMEM((NUM_CLASSES, _CB), jnp.float32),
            pltpu.SemaphoreType.DMA,
            pltpu.SemaphoreType.DMA,
            pltpu.SemaphoreType.DMA,
            pltpu.SemaphoreType.DMA,
        ],
        compiler_params=pltpu.CompilerParams(
            use_tc_tiling_on_sc=False, needs_layout_passes=False
        ),
    )
    def gather_k(
        ids_hbm, p_hbm, out_hbm,
        idx_v, hbuf0, hbuf1, gbuf0, gbuf1, skew0, skew1, obuf0, obuf1,
        gsem0, gsem1, ssem0, ssem1,
    ):
        wid = lax.axis_index("s") * nc + lax.axis_index("c")
        gbase = wid * cpw
        pltpu.sync_copy(ids_hbm.at[pl.ds(gbase, cpw)], idx_v)

        def halve(j, hbuf):
            for k in range(_CB // 16):
                hbuf[pl.ds(16 * k, 16)] = idx_v[j, pl.ds(16 * k, 16)] >> 1

        halve(0, hbuf0)
        halve(1, hbuf1)
        pltpu.async_copy(p_hbm.at[hbuf0], gbuf0, gsem0)
        pltpu.async_copy(p_hbm.at[hbuf1], gbuf1, gsem1)

        iota16 = lax.iota(jnp.int32, 16)

        def skew_pass(gbuf, skew):
            def rowcopy(i, c):
                off = i & 15
                for k in range(2 * NUM_CLASSES // 16):
                    skew[i, pl.ds(off + 16 * k, 16)] = gbuf[
                        i, pl.ds(16 * k, 16)
                    ]
                return c

            lax.fori_loop(0, _CB, rowcopy, 0, unroll=8)

        def transpose_pass(j, skew, obuf):
            for k in range(_CB // 16):
                rows = iota16 + 16 * k
                base = iota16 + (idx_v[j, pl.ds(16 * k, 16)] & 1) * NUM_CLASSES
                for cg in range(0, NUM_CLASSES, 8):
                    vals = [
                        plsc.load_gather(skew, [rows, base + (cg + t)])
                        for t in range(8)
                    ]
                    for t in range(8):
                        obuf[cg + t, pl.ds(16 * k, 16)] = vals[t]

        def out_slice(j):
            g = gbase + j
            return out_hbm.at[g // nbc, :, pl.ds((g % nbc) * _CB, _CB)]

        def half_step(i, j, gbuf, skew, obuf, hbuf, gsem, ssem):
            pltpu.make_async_copy(p_hbm.at[hbuf], gbuf, gsem).wait()
            skew_pass(gbuf, skew)

            @pl.when(j + 2 < cpw)
            def _():
                halve(j + 2, hbuf)
                pltpu.async_copy(p_hbm.at[hbuf], gbuf, gsem)

            @pl.when(i > 0)
            def _():
                pltpu.make_async_copy(obuf, out_slice(j - 2), ssem).wait()

            transpose_pass(j, skew, obuf)
            pltpu.async_copy(obuf, out_slice(j), ssem)

        def body(i, carry):
            j = 2 * i
            half_step(i, j, gbuf0, skew0, obuf0, hbuf0, gsem0, ssem0)
            half_step(i, j + 1, gbuf1, skew1, obuf1, hbuf1, gsem1, ssem1)
            return carry

        lax.fori_loop(0, cpw // 2, body, 0)
        pltpu.make_async_copy(obuf0, out_slice(cpw - 2), ssem0).wait()
        pltpu.make_async_copy(obuf1, out_slice(cpw - 1), ssem1).wait()

    return gather_k


def kernel(token_ids, emb_table, W, b):
    B, L = token_ids.shape
    info = plsc.get_sparse_core_info()
    nw = info.num_cores * info.num_subcores

    emb2 = emb_table.reshape(VOCAB // 2, 2 * EMB_DIM)
    w_blk = (
        jnp.zeros((2 * EMB_DIM, 2 * NUM_CLASSES), jnp.float32)
        .at[:EMB_DIM, :NUM_CLASSES].set(W)
        .at[EMB_DIM:, NUM_CLASSES:].set(W)
    )
    b2 = jnp.concatenate([b, b]).reshape(1, 2 * NUM_CLASSES)
    proj = _project(emb2, w_blk, b2)

    ids2 = token_ids.T.reshape((B // _CB) * L, _CB).astype(jnp.int32)
    out3 = _make_gather(B, L, nw)(ids2, proj)
    return out3.transpose(2, 0, 1)


# speedup vs baseline: 1.5831x; 1.5831x over previous
import functools

import jax
import jax.numpy as jnp
from jax import lax
from jax.experimental import pallas as pl
from jax.experimental.pallas import tpu as pltpu
from jax.experimental.pallas import tpu_sc as plsc

VOCAB = 100000
EMB_DIM = 128
NUM_CLASSES = 64


_ROWS_PER_BLOCK = 4000


def _project_body(emb_ref, w_ref, b_ref, out_ref):
    out_ref[...] = (
        jnp.dot(emb_ref[...], w_ref[...], preferred_element_type=jnp.float32)
        + b_ref[...]
    )


def _project(emb_table, W, b2):
    n_blocks = VOCAB // _ROWS_PER_BLOCK
    return pl.pallas_call(
        _project_body,
        grid=(n_blocks,),
        in_specs=[
            pl.BlockSpec((_ROWS_PER_BLOCK, EMB_DIM), lambda i: (i, 0)),
            pl.BlockSpec((EMB_DIM, NUM_CLASSES), lambda i: (0, 0)),
            pl.BlockSpec((1, NUM_CLASSES), lambda i: (0, 0)),
        ],
        out_specs=pl.BlockSpec((_ROWS_PER_BLOCK, NUM_CLASSES), lambda i: (i, 0)),
        out_shape=jax.ShapeDtypeStruct((VOCAB, NUM_CLASSES), jnp.float32),
    )(emb_table, W, b2)


_CHUNK = 40


def _make_gather(B, L, nw):
    n_ids = B * L
    ids_per_w = n_ids // nw
    n_chunks = ids_per_w // _CHUNK
    assert n_chunks % 2 == 0
    cpb = L // _CHUNK
    row_w = L * NUM_CLASSES
    seg = _CHUNK * NUM_CLASSES
    mesh = plsc.VectorSubcoreMesh(core_axis_name="c", subcore_axis_name="s")
    nc = mesh.num_cores

    @functools.partial(
        pl.kernel,
        mesh=mesh,
        out_type=jax.ShapeDtypeStruct((B, row_w), jnp.float32),
        scratch_types=[
            pltpu.VMEM((ids_per_w,), jnp.int32),
            pltpu.VMEM((_CHUNK, NUM_CLASSES), jnp.float32),
            pltpu.VMEM((_CHUNK, NUM_CLASSES), jnp.float32),
            pltpu.VMEM((seg,), jnp.float32),
            pltpu.VMEM((seg,), jnp.float32),
            pltpu.SemaphoreType.DMA,
            pltpu.SemaphoreType.DMA,
            pltpu.SemaphoreType.DMA,
            pltpu.SemaphoreType.DMA,
        ],
        compiler_params=pltpu.CompilerParams(
            use_tc_tiling_on_sc=False, needs_layout_passes=False
        ),
    )
    def gather_k(
        ids_hbm, p_hbm, out_hbm,
        idx_v, gbuf0, gbuf1, obuf0, obuf1, gsem0, gsem1, ssem0, ssem1,
    ):
        wid = lax.axis_index("s") * nc + lax.axis_index("c")
        pltpu.sync_copy(ids_hbm.at[wid], idx_v)

        def idxs(j):
            return idx_v.at[pl.ds(_CHUNK * j, _CHUNK)]

        pltpu.async_copy(p_hbm.at[idxs(0)], gbuf0, gsem0)
        pltpu.async_copy(p_hbm.at[idxs(1)], gbuf1, gsem1)

        def linearize(gbuf, obuf):
            def rowc(r, c):
                for k in range(NUM_CLASSES // 16):
                    obuf[pl.ds(r * NUM_CLASSES + 16 * k, 16)] = gbuf[
                        r, pl.ds(16 * k, 16)
                    ]
                return c

            lax.fori_loop(0, _CHUNK, rowc, 0, unroll=8)

        def out_slice(j):
            g = wid * n_chunks + j
            return out_hbm.at[g // cpb, pl.ds((g % cpb) * seg, seg)]

        def half_step(i, j, gbuf, obuf, gsem, ssem):
            pltpu.make_async_copy(p_hbm.at[idxs(j)], gbuf, gsem).wait()

            @pl.when(i > 0)
            def _():
                pltpu.make_async_copy(obuf, out_slice(j - 2), ssem).wait()

            linearize(gbuf, obuf)

            @pl.when(j + 2 < n_chunks)
            def _():
                pltpu.async_copy(p_hbm.at[idxs(j + 2)], gbuf, gsem)

            pltpu.async_copy(obuf, out_slice(j), ssem)

        def body(i, carry):
            j = 2 * i
            half_step(i, j, gbuf0, obuf0, gsem0, ssem0)
            half_step(i, j + 1, gbuf1, obuf1, gsem1, ssem1)
            return carry

        lax.fori_loop(0, n_chunks // 2, body, 0)
        pltpu.make_async_copy(obuf0, out_slice(n_chunks - 2), ssem0).wait()
        pltpu.make_async_copy(obuf1, out_slice(n_chunks - 1), ssem1).wait()

    return gather_k


_BT = 512


def _transpose_body(x_ref, out_ref):
    out_ref[...] = x_ref[...].T


def _transpose(x2d, B, row_w):
    return pl.pallas_call(
        _transpose_body,
        grid=(B // _BT, row_w // _BT),
        in_specs=[pl.BlockSpec((_BT, _BT), lambda i, j: (i, j))],
        out_specs=pl.BlockSpec((_BT, _BT), lambda i, j: (j, i)),
        out_shape=jax.ShapeDtypeStruct((row_w, B), jnp.float32),
    )(x2d)


def kernel(token_ids, emb_table, W, b):
    B, L = token_ids.shape
    info = plsc.get_sparse_core_info()
    nw = info.num_cores * info.num_subcores

    proj = _project(emb_table, W, b.reshape(1, NUM_CLASSES))

    ids2 = token_ids.reshape(nw, (B // nw) * L).astype(jnp.int32)
    out2 = _make_gather(B, L, nw)(ids2, proj)
    out2t = _transpose(out2, B, L * NUM_CLASSES)
    return out2t.reshape(L, NUM_CLASSES, B).transpose(2, 0, 1)


# speedup vs baseline: 1.7588x; 1.1110x over previous
import functools

import jax
import jax.numpy as jnp
from jax import lax
from jax.experimental import pallas as pl
from jax.experimental.pallas import tpu as pltpu
from jax.experimental.pallas import tpu_sc as plsc

VOCAB = 100000
EMB_DIM = 128
NUM_CLASSES = 64


_ROWS_PER_BLOCK = 4000


def _project_body(emb_ref, w_ref, b_ref, out_ref):
    out_ref[...] = (
        jnp.dot(emb_ref[...], w_ref[...], preferred_element_type=jnp.float32)
        + b_ref[...]
    )


def _project(emb_table, W, b2):
    n_blocks = VOCAB // _ROWS_PER_BLOCK
    return pl.pallas_call(
        _project_body,
        grid=(n_blocks,),
        in_specs=[
            pl.BlockSpec((_ROWS_PER_BLOCK, EMB_DIM), lambda i: (i, 0)),
            pl.BlockSpec((EMB_DIM, NUM_CLASSES), lambda i: (0, 0)),
            pl.BlockSpec((1, NUM_CLASSES), lambda i: (0, 0)),
        ],
        out_specs=pl.BlockSpec((_ROWS_PER_BLOCK, NUM_CLASSES), lambda i: (i, 0)),
        out_shape=jax.ShapeDtypeStruct((VOCAB, NUM_CLASSES), jnp.float32),
    )(emb_table, W, b2)


_CHUNK = 40


def _make_gather(B, L, nw):
    n_ids = B * L
    ids_per_w = n_ids // nw
    n_chunks = ids_per_w // _CHUNK
    assert n_chunks % 2 == 0
    cpb = L // _CHUNK
    row_w = L * NUM_CLASSES
    seg = _CHUNK * NUM_CLASSES
    mesh = plsc.VectorSubcoreMesh(core_axis_name="c", subcore_axis_name="s")
    nc = mesh.num_cores

    _RING = 8
    _AHEAD = 4
    assert n_chunks % _RING == 0

    @functools.partial(
        pl.kernel,
        mesh=mesh,
        out_type=jax.ShapeDtypeStruct((B, L, NUM_CLASSES), jnp.float32),
        scratch_types=[
            pltpu.VMEM((ids_per_w,), jnp.int32),
        ]
        + [pltpu.VMEM((_CHUNK, NUM_CLASSES), jnp.float32)] * _RING
        + [pltpu.SemaphoreType.DMA] * (2 * _RING),
        compiler_params=pltpu.CompilerParams(
            use_tc_tiling_on_sc=False, needs_layout_passes=False
        ),
    )
    def gather_k(ids_hbm, p_hbm, out_hbm, idx_v, *bufs_and_sems):
        gbuf = bufs_and_sems[:_RING]
        gsem = bufs_and_sems[_RING : 2 * _RING]
        ssem = bufs_and_sems[2 * _RING :]
        wid = lax.axis_index("s") * nc + lax.axis_index("c")
        pltpu.sync_copy(ids_hbm.at[wid], idx_v)

        def idxs(j):
            return idx_v.at[pl.ds(_CHUNK * j, _CHUNK)]

        def out_slice(j):
            g = wid * n_chunks + j
            return out_hbm.at[g // cpb, pl.ds((g % cpb) * _CHUNK, _CHUNK)]

        for t in range(_AHEAD):
            pltpu.async_copy(p_hbm.at[idxs(t)], gbuf[t], gsem[t])

        def body(i, carry):
            j0 = _RING * i
            for t in range(_RING):
                j = j0 + t
                bn = (t + _AHEAD) % _RING

                @pl.when(j + _AHEAD < n_chunks)
                def _(j=j, bn=bn):
                    @pl.when(j + _AHEAD >= _RING)
                    def _():
                        pltpu.make_async_copy(
                            gbuf[bn], out_slice(j + _AHEAD - _RING), ssem[bn]
                        ).wait()

                    pltpu.async_copy(
                        p_hbm.at[idxs(j + _AHEAD)], gbuf[bn], gsem[bn]
                    )

                pltpu.make_async_copy(p_hbm.at[idxs(j)], gbuf[t], gsem[t]).wait()
                pltpu.async_copy(gbuf[t], out_slice(j), ssem[t])
            return carry

        lax.fori_loop(0, n_chunks // _RING, body, 0)
        for t in range(_RING):
            j = n_chunks - _RING + t
            pltpu.make_async_copy(gbuf[t], out_slice(j), ssem[t]).wait()

    return gather_k


_BT = 512


def _transpose_body(x_ref, out_ref):
    out_ref[...] = x_ref[...].T


def _transpose(x2d, B, row_w):
    return pl.pallas_call(
        _transpose_body,
        grid=(B // _BT, row_w // _BT),
        in_specs=[pl.BlockSpec((_BT, _BT), lambda i, j: (i, j))],
        out_specs=pl.BlockSpec((_BT, _BT), lambda i, j: (j, i)),
        out_shape=jax.ShapeDtypeStruct((row_w, B), jnp.float32),
    )(x2d)


def kernel(token_ids, emb_table, W, b):
    B, L = token_ids.shape
    info = plsc.get_sparse_core_info()
    nw = info.num_cores * info.num_subcores

    proj = _project(emb_table, W, b.reshape(1, NUM_CLASSES))

    ids2 = token_ids.reshape(nw, (B // nw) * L).astype(jnp.int32)
    return _make_gather(B, L, nw)(ids2, proj)


# speedup vs baseline: 2.1160x; 1.2031x over previous
import functools

import jax
import jax.numpy as jnp
from jax import lax
from jax.experimental import pallas as pl
from jax.experimental.pallas import tpu as pltpu
from jax.experimental.pallas import tpu_sc as plsc

VOCAB = 100000
EMB_DIM = 128
NUM_CLASSES = 64


_ROWS_PER_BLOCK = 4000


def _project_body(emb_ref, w_ref, b_ref, out_ref):
    out_ref[...] = (
        jnp.dot(emb_ref[...], w_ref[...], preferred_element_type=jnp.float32)
        + b_ref[...]
    )


def _project(emb_table, W, b2):
    n_blocks = VOCAB // _ROWS_PER_BLOCK
    return pl.pallas_call(
        _project_body,
        grid=(n_blocks,),
        in_specs=[
            pl.BlockSpec((_ROWS_PER_BLOCK, EMB_DIM), lambda i: (i, 0)),
            pl.BlockSpec((EMB_DIM, NUM_CLASSES), lambda i: (0, 0)),
            pl.BlockSpec((1, NUM_CLASSES), lambda i: (0, 0)),
        ],
        out_specs=pl.BlockSpec((_ROWS_PER_BLOCK, NUM_CLASSES), lambda i: (i, 0)),
        out_shape=jax.ShapeDtypeStruct((VOCAB, NUM_CLASSES), jnp.float32),
    )(emb_table, W, b2)


_CHUNK = 40


def _make_gather(B, L, nw):
    n_ids = B * L
    ids_per_w = n_ids // nw
    n_chunks = ids_per_w // _CHUNK
    assert n_chunks % 2 == 0
    cpb = L // _CHUNK
    row_w = L * NUM_CLASSES
    seg = _CHUNK * NUM_CLASSES
    mesh = plsc.VectorSubcoreMesh(core_axis_name="c", subcore_axis_name="s")
    nc = mesh.num_cores

    _RING = 8
    _AHEAD = 4
    assert n_chunks % _RING == 0

    @functools.partial(
        pl.kernel,
        mesh=mesh,
        out_type=jax.ShapeDtypeStruct((B, row_w), jnp.float32),
        scratch_types=[
            pltpu.VMEM((ids_per_w,), jnp.int32),
        ]
        + [pltpu.VMEM((_CHUNK, NUM_CLASSES), jnp.float32)] * _RING
        + [pltpu.VMEM((seg,), jnp.float32)] * _RING
        + [pltpu.SemaphoreType.DMA] * (2 * _RING),
        compiler_params=pltpu.CompilerParams(
            use_tc_tiling_on_sc=False, needs_layout_passes=False
        ),
    )
    def gather_k(ids_hbm, p_hbm, out_hbm, idx_v, *bufs_and_sems):
        gbuf = bufs_and_sems[:_RING]
        obuf = bufs_and_sems[_RING : 2 * _RING]
        gsem = bufs_and_sems[2 * _RING : 3 * _RING]
        ssem = bufs_and_sems[3 * _RING :]
        wid = lax.axis_index("s") * nc + lax.axis_index("c")
        pltpu.sync_copy(ids_hbm.at[wid], idx_v)

        def idxs(j):
            return idx_v.at[pl.ds(_CHUNK * j, _CHUNK)]

        def out_slice(j):
            g = wid * n_chunks + j
            return out_hbm.at[g // cpb, pl.ds((g % cpb) * seg, seg)]

        def linearize(gbuf_t, obuf_t):
            def rowc(r, c):
                for k in range(NUM_CLASSES // 16):
                    obuf_t[pl.ds(r * NUM_CLASSES + 16 * k, 16)] = gbuf_t[
                        r, pl.ds(16 * k, 16)
                    ]
                return c

            lax.fori_loop(0, _CHUNK, rowc, 0, unroll=8)

        for t in range(_AHEAD):
            pltpu.async_copy(p_hbm.at[idxs(t)], gbuf[t], gsem[t])

        def body(i, carry):
            j0 = _RING * i
            for t in range(_RING):
                j = j0 + t
                bn = (t + _AHEAD) % _RING

                pltpu.make_async_copy(p_hbm.at[idxs(j)], gbuf[t], gsem[t]).wait()

                @pl.when(j + _AHEAD < n_chunks)
                def _(j=j, bn=bn):
                    pltpu.async_copy(
                        p_hbm.at[idxs(j + _AHEAD)], gbuf[bn], gsem[bn]
                    )

                @pl.when(j >= _RING)
                def _(j=j, t=t):
                    pltpu.make_async_copy(
                        obuf[t], out_slice(j - _RING), ssem[t]
                    ).wait()

                linearize(gbuf[t], obuf[t])
                pltpu.async_copy(obuf[t], out_slice(j), ssem[t])
            return carry

        lax.fori_loop(0, n_chunks // _RING, body, 0)
        for t in range(_RING):
            j = n_chunks - _RING + t
            pltpu.make_async_copy(obuf[t], out_slice(j), ssem[t]).wait()

    return gather_k


_BT = 512


def _transpose_body(x_ref, out_ref):
    out_ref[...] = x_ref[...].T


def _transpose(x2d, B, row_w):
    return pl.pallas_call(
        _transpose_body,
        grid=(B // _BT, row_w // _BT),
        in_specs=[pl.BlockSpec((_BT, _BT), lambda i, j: (i, j))],
        out_specs=pl.BlockSpec((_BT, _BT), lambda i, j: (j, i)),
        out_shape=jax.ShapeDtypeStruct((row_w, B), jnp.float32),
    )(x2d)


def kernel(token_ids, emb_table, W, b):
    B, L = token_ids.shape
    info = plsc.get_sparse_core_info()
    nw = info.num_cores * info.num_subcores

    proj = _project(emb_table, W, b.reshape(1, NUM_CLASSES))

    ids2 = token_ids.reshape(nw, (B // nw) * L).astype(jnp.int32)
    out2 = _make_gather(B, L, nw)(ids2, proj)
    return out2.T.reshape(L, NUM_CLASSES, B).transpose(2, 0, 1)


# speedup vs baseline: 2.1941x; 1.0369x over previous
import functools

import jax
import jax.numpy as jnp
from jax import lax
from jax.experimental import pallas as pl
from jax.experimental.pallas import tpu as pltpu
from jax.experimental.pallas import tpu_sc as plsc

VOCAB = 100000
EMB_DIM = 128
NUM_CLASSES = 64


_ROWS_PER_BLOCK = 4000


def _project_body(emb_ref, w_ref, b_ref, out_ref):
    out_ref[...] = (
        jnp.dot(emb_ref[...], w_ref[...], preferred_element_type=jnp.float32)
        + b_ref[...]
    )


def _project(emb_table, W, b2):
    n_blocks = VOCAB // _ROWS_PER_BLOCK
    return pl.pallas_call(
        _project_body,
        grid=(n_blocks,),
        in_specs=[
            pl.BlockSpec((_ROWS_PER_BLOCK, EMB_DIM), lambda i: (i, 0)),
            pl.BlockSpec((EMB_DIM, 2 * NUM_CLASSES), lambda i: (0, 0)),
            pl.BlockSpec((1, 2 * NUM_CLASSES), lambda i: (0, 0)),
        ],
        out_specs=pl.BlockSpec(
            (_ROWS_PER_BLOCK, 2 * NUM_CLASSES), lambda i: (i, 0)
        ),
        out_shape=jax.ShapeDtypeStruct((VOCAB, 2 * NUM_CLASSES), jnp.float32),
    )(emb_table, W, b2)


_CHUNK = 40


def _make_gather(B, L, nw):
    n_ids = B * L
    ids_per_w = n_ids // nw
    n_chunks = ids_per_w // _CHUNK
    assert n_chunks % 2 == 0
    cpb = L // _CHUNK
    row_w = L * NUM_CLASSES
    seg = _CHUNK * NUM_CLASSES
    mesh = plsc.VectorSubcoreMesh(core_axis_name="c", subcore_axis_name="s")
    nc = mesh.num_cores

    _RING = 8
    _AHEAD = 4
    assert n_chunks % _RING == 0

    @functools.partial(
        pl.kernel,
        mesh=mesh,
        out_type=jax.ShapeDtypeStruct((B, row_w), jnp.float32),
        scratch_types=[
            pltpu.VMEM((ids_per_w,), jnp.int32),
        ]
        + [pltpu.VMEM((_CHUNK, 2 * NUM_CLASSES), jnp.float32)] * _RING
        + [pltpu.VMEM((seg,), jnp.float32)] * _RING
        + [pltpu.SemaphoreType.DMA] * (2 * _RING),
    )
    def gather_k(ids_hbm, p_hbm, out_hbm, idx_v, *bufs_and_sems):
        gbuf = bufs_and_sems[:_RING]
        obuf = bufs_and_sems[_RING : 2 * _RING]
        gsem = bufs_and_sems[2 * _RING : 3 * _RING]
        ssem = bufs_and_sems[3 * _RING :]
        wid = lax.axis_index("s") * nc + lax.axis_index("c")
        pltpu.sync_copy(ids_hbm.at[wid], idx_v)

        def idxs(j):
            return idx_v.at[pl.ds(_CHUNK * j, _CHUNK)]

        def out_slice(j):
            g = wid * n_chunks + j
            return out_hbm.at[g // cpb, pl.ds((g % cpb) * seg, seg)]

        def linearize(gbuf_t, obuf_t):
            def rowc(r, c):
                for k in range(NUM_CLASSES // 16):
                    obuf_t[pl.ds(r * NUM_CLASSES + 16 * k, 16)] = gbuf_t[
                        r, pl.ds(16 * k, 16)
                    ]
                return c

            lax.fori_loop(0, _CHUNK, rowc, 0, unroll=8)

        for t in range(_AHEAD):
            pltpu.async_copy(p_hbm.at[idxs(t)], gbuf[t], gsem[t])

        def body(i, carry):
            j0 = _RING * i
            for t in range(_RING):
                j = j0 + t
                bn = (t + _AHEAD) % _RING

                pltpu.make_async_copy(p_hbm.at[idxs(j)], gbuf[t], gsem[t]).wait()

                @pl.when(j + _AHEAD < n_chunks)
                def _(j=j, bn=bn):
                    pltpu.async_copy(
                        p_hbm.at[idxs(j + _AHEAD)], gbuf[bn], gsem[bn]
                    )

                @pl.when(j >= _RING)
                def _(j=j, t=t):
                    pltpu.make_async_copy(
                        obuf[t], out_slice(j - _RING), ssem[t]
                    ).wait()

                linearize(gbuf[t], obuf[t])
                pltpu.async_copy(obuf[t], out_slice(j), ssem[t])
            return carry

        lax.fori_loop(0, n_chunks // _RING, body, 0)
        for t in range(_RING):
            j = n_chunks - _RING + t
            pltpu.make_async_copy(obuf[t], out_slice(j), ssem[t]).wait()

    return gather_k


_BT = 512


def _transpose_body(x_ref, out_ref):
    out_ref[...] = x_ref[...].T


def _transpose(x2d, B, row_w):
    return pl.pallas_call(
        _transpose_body,
        grid=(B // _BT, row_w // _BT),
        in_specs=[pl.BlockSpec((_BT, _BT), lambda i, j: (i, j))],
        out_specs=pl.BlockSpec((_BT, _BT), lambda i, j: (j, i)),
        out_shape=jax.ShapeDtypeStruct((row_w, B), jnp.float32),
    )(x2d)


def kernel(token_ids, emb_table, W, b):
    B, L = token_ids.shape
    info = plsc.get_sparse_core_info()
    nw = info.num_cores * info.num_subcores

    W2 = jnp.concatenate([W, W], axis=1)
    b2 = jnp.concatenate([b, b]).reshape(1, 2 * NUM_CLASSES)
    proj = _project(emb_table, W2, b2)

    ids2 = token_ids.reshape(nw, (B // nw) * L).astype(jnp.int32)
    out2 = _make_gather(B, L, nw)(ids2, proj)
    return out2.T.reshape(L, NUM_CLASSES, B).transpose(2, 0, 1)


# speedup vs baseline: 2.5610x; 1.1672x over previous
import functools

import jax
import jax.numpy as jnp
from jax import lax
from jax.experimental import pallas as pl
from jax.experimental.pallas import tpu as pltpu
from jax.experimental.pallas import tpu_sc as plsc

VOCAB = 100000
EMB_DIM = 128
NUM_CLASSES = 64


_ROWS_PER_BLOCK = 4000


def _project_body(emb_ref, w_ref, b_ref, out_ref):
    out_ref[...] = (
        jnp.dot(emb_ref[...], w_ref[...], preferred_element_type=jnp.float32)
        + b_ref[...]
    )


def _project(emb_table, W, b2):
    n_blocks = VOCAB // _ROWS_PER_BLOCK
    return pl.pallas_call(
        _project_body,
        grid=(n_blocks,),
        in_specs=[
            pl.BlockSpec((_ROWS_PER_BLOCK, EMB_DIM), lambda i: (i, 0)),
            pl.BlockSpec((EMB_DIM, 2 * NUM_CLASSES), lambda i: (0, 0)),
            pl.BlockSpec((1, 2 * NUM_CLASSES), lambda i: (0, 0)),
        ],
        out_specs=pl.BlockSpec(
            (_ROWS_PER_BLOCK, 2 * NUM_CLASSES), lambda i: (i, 0)
        ),
        out_shape=jax.ShapeDtypeStruct((VOCAB, 2 * NUM_CLASSES), jnp.float32),
    )(emb_table, W, b2)


_BG = 8
_LC = 20
_TOK = _BG * _LC


def _make_gather(B, L, nw):
    n_ids = B * L
    ids_per_w = n_ids // nw
    n_chunks = ids_per_w // _TOK
    assert n_chunks % 2 == 0
    lpb = L // _LC
    row_w = L * NUM_CLASSES
    seg = _LC * NUM_CLASSES
    mesh = plsc.VectorSubcoreMesh(core_axis_name="c", subcore_axis_name="s")
    nc = mesh.num_cores

    @functools.partial(
        pl.kernel,
        mesh=mesh,
        out_type=jax.ShapeDtypeStruct((B, row_w), jnp.float32),
        scratch_types=[
            pltpu.VMEM((ids_per_w,), jnp.int32),
            pltpu.VMEM((_TOK, 2 * NUM_CLASSES), jnp.float32),
            pltpu.VMEM((_TOK, 2 * NUM_CLASSES), jnp.float32),
            pltpu.VMEM((_BG, seg), jnp.float32),
            pltpu.VMEM((_BG, seg), jnp.float32),
            pltpu.SemaphoreType.DMA,
            pltpu.SemaphoreType.DMA,
            pltpu.SemaphoreType.DMA,
            pltpu.SemaphoreType.DMA,
        ],
    )
    def gather_k(
        ids_hbm, p_hbm, out_hbm,
        idx_v, gbuf0, gbuf1, obuf0, obuf1, gsem0, gsem1, ssem0, ssem1,
    ):
        wid = lax.axis_index("s") * nc + lax.axis_index("c")
        pltpu.sync_copy(ids_hbm.at[wid], idx_v)

        def start_gather(j, gbuf, gsem):
            pltpu.async_copy(
                p_hbm.at[idx_v.at[pl.ds(_TOK * j, 128)]],
                gbuf.at[pl.ds(0, 128)], gsem,
            )
            pltpu.async_copy(
                p_hbm.at[idx_v.at[pl.ds(_TOK * j + 128, _TOK - 128)]],
                gbuf.at[pl.ds(128, _TOK - 128)], gsem,
            )

        def wait_gather(j, gbuf, gsem):
            pltpu.make_async_copy(
                p_hbm.at[idx_v.at[pl.ds(_TOK * j, 128)]],
                gbuf.at[pl.ds(0, 128)], gsem,
            ).wait()
            pltpu.make_async_copy(
                p_hbm.at[idx_v.at[pl.ds(_TOK * j + 128, _TOK - 128)]],
                gbuf.at[pl.ds(128, _TOK - 128)], gsem,
            ).wait()

        def out_slice(j):
            g = wid * n_chunks + j
            b0 = (g // lpb) * _BG
            return out_hbm.at[
                pl.ds(b0, _BG), pl.ds((g % lpb) * seg, seg)
            ]

        def linearize(gbuf_t, obuf_t):
            for bi in range(_BG):
                def rowc(dl, c, bi=bi):
                    for k in range(NUM_CLASSES // 16):
                        obuf_t[bi, pl.ds(dl * NUM_CLASSES + 16 * k, 16)] = (
                            gbuf_t[bi * _LC + dl, pl.ds(16 * k, 16)]
                        )
                    return c

                lax.fori_loop(0, _LC, rowc, 0, unroll=5)

        start_gather(0, gbuf0, gsem0)
        start_gather(1, gbuf1, gsem1)

        def half_step(i, j, gbuf, obuf, gsem, ssem):
            wait_gather(j, gbuf, gsem)

            @pl.when(i > 0)
            def _():
                pltpu.make_async_copy(obuf, out_slice(j - 2), ssem).wait()

            linearize(gbuf, obuf)

            @pl.when(j + 2 < n_chunks)
            def _():
                start_gather(j + 2, gbuf, gsem)

            pltpu.async_copy(obuf, out_slice(j), ssem)

        def body(i, carry):
            j = 2 * i
            half_step(i, j, gbuf0, obuf0, gsem0, ssem0)
            half_step(i, j + 1, gbuf1, obuf1, gsem1, ssem1)
            return carry

        lax.fori_loop(0, n_chunks // 2, body, 0)
        pltpu.make_async_copy(obuf0, out_slice(n_chunks - 2), ssem0).wait()
        pltpu.make_async_copy(obuf1, out_slice(n_chunks - 1), ssem1).wait()

    return gather_k


_BT = 512


def _transpose_body(x_ref, out_ref):
    out_ref[...] = x_ref[...].T


def _transpose(x2d, B, row_w):
    return pl.pallas_call(
        _transpose_body,
        grid=(B // _BT, row_w // _BT),
        in_specs=[pl.BlockSpec((_BT, _BT), lambda i, j: (i, j))],
        out_specs=pl.BlockSpec((_BT, _BT), lambda i, j: (j, i)),
        out_shape=jax.ShapeDtypeStruct((row_w, B), jnp.float32),
    )(x2d)


def kernel(token_ids, emb_table, W, b):
    B, L = token_ids.shape
    info = plsc.get_sparse_core_info()
    nw = info.num_cores * info.num_subcores

    W2 = jnp.concatenate([W, W], axis=1)
    b2 = jnp.concatenate([b, b]).reshape(1, 2 * NUM_CLASSES)
    proj = _project(emb_table, W2, b2)

    ids2 = (
        token_ids.reshape(nw, (B // nw) // _BG, _BG, L // _LC, _LC)
        .transpose(0, 1, 3, 2, 4)
        .reshape(nw, (B // nw) * L)
        .astype(jnp.int32)
    )
    out2 = _make_gather(B, L, nw)(ids2, proj)
    return out2.T.reshape(L, NUM_CLASSES, B).transpose(2, 0, 1)


# speedup vs baseline: 2.5615x; 1.0002x over previous
import functools

import jax
import jax.numpy as jnp
from jax import lax
from jax.experimental import pallas as pl
from jax.experimental.pallas import tpu as pltpu
from jax.experimental.pallas import tpu_sc as plsc

VOCAB = 100000
EMB_DIM = 128
NUM_CLASSES = 64


_ROWS_PER_BLOCK = 4000


def _project_body(emb_ref, w_ref, b_ref, out_ref):
    out_ref[...] = (
        jnp.dot(emb_ref[...], w_ref[...], preferred_element_type=jnp.float32)
        + b_ref[...]
    )


def _project(emb_table, W, b2):
    n_blocks = VOCAB // _ROWS_PER_BLOCK
    return pl.pallas_call(
        _project_body,
        grid=(n_blocks,),
        in_specs=[
            pl.BlockSpec((_ROWS_PER_BLOCK, EMB_DIM), lambda i: (i, 0)),
            pl.BlockSpec((EMB_DIM, 2 * NUM_CLASSES), lambda i: (0, 0)),
            pl.BlockSpec((1, 2 * NUM_CLASSES), lambda i: (0, 0)),
        ],
        out_specs=pl.BlockSpec(
            (_ROWS_PER_BLOCK, 2 * NUM_CLASSES), lambda i: (i, 0)
        ),
        out_shape=jax.ShapeDtypeStruct((VOCAB, 2 * NUM_CLASSES), jnp.float32),
    )(emb_table, W, b2)


_BG = 8
_LC = 20
_TOK = _BG * _LC


def _make_gather(B, L, nw):
    n_ids = B * L
    ids_per_w = n_ids // nw
    n_chunks = ids_per_w // _TOK
    assert n_chunks % 2 == 0
    lpb = L // _LC
    row_w = L * NUM_CLASSES
    seg = _LC * NUM_CLASSES
    mesh = plsc.VectorSubcoreMesh(core_axis_name="c", subcore_axis_name="s")
    nc = mesh.num_cores

    @functools.partial(
        pl.kernel,
        mesh=mesh,
        out_type=jax.ShapeDtypeStruct((B, row_w), jnp.float32),
        scratch_types=[
            pltpu.VMEM((ids_per_w,), jnp.int32),
            pltpu.VMEM((_TOK, 2 * NUM_CLASSES), jnp.float32),
            pltpu.VMEM((_TOK, 2 * NUM_CLASSES), jnp.float32),
            pltpu.VMEM((_BG, seg), jnp.float32),
            pltpu.VMEM((_BG, seg), jnp.float32),
            pltpu.SemaphoreType.DMA,
            pltpu.SemaphoreType.DMA,
            pltpu.SemaphoreType.DMA,
            pltpu.SemaphoreType.DMA,
        ],
    )
    def gather_k(
        ids_hbm, p_hbm, out_hbm,
        idx_v, gbuf0, gbuf1, obuf0, obuf1, gsem0, gsem1, ssem0, ssem1,
    ):
        wid = lax.axis_index("s") * nc + lax.axis_index("c")
        pltpu.sync_copy(ids_hbm.at[wid], idx_v)

        def start_gather(j, gbuf, gsem):
            pltpu.async_copy(
                p_hbm.at[idx_v.at[pl.ds(_TOK * j, 128)]],
                gbuf.at[pl.ds(0, 128)], gsem,
            )
            pltpu.async_copy(
                p_hbm.at[idx_v.at[pl.ds(_TOK * j + 128, _TOK - 128)]],
                gbuf.at[pl.ds(128, _TOK - 128)], gsem,
            )

        def wait_gather(j, gbuf, gsem):
            pltpu.make_async_copy(
                p_hbm.at[idx_v.at[pl.ds(_TOK * j, 128)]],
                gbuf.at[pl.ds(0, 128)], gsem,
            ).wait()
            pltpu.make_async_copy(
                p_hbm.at[idx_v.at[pl.ds(_TOK * j + 128, _TOK - 128)]],
                gbuf.at[pl.ds(128, _TOK - 128)], gsem,
            ).wait()

        def out_slice(j):
            g = wid * n_chunks + j
            b0 = (g // lpb) * _BG
            return out_hbm.at[
                pl.ds(b0, _BG), pl.ds((g % lpb) * seg, seg)
            ]

        def linearize(gbuf_t, obuf_t):
            for bi in range(_BG):
                def rowc(dl, c, bi=bi):
                    for k in range(NUM_CLASSES // 16):
                        obuf_t[bi, pl.ds(dl * NUM_CLASSES + 16 * k, 16)] = (
                            gbuf_t[bi * _LC + dl, pl.ds(16 * k, 16)]
                        )
                    return c

                lax.fori_loop(0, _LC, rowc, 0, unroll=5)

        start_gather(0, gbuf0, gsem0)
        start_gather(1, gbuf1, gsem1)

        def half_step(i, j, gbuf, obuf, gsem, ssem):
            wait_gather(j, gbuf, gsem)

            @pl.when(i > 0)
            def _():
                pltpu.make_async_copy(obuf, out_slice(j - 2), ssem).wait()

            linearize(gbuf, obuf)

            @pl.when(j + 2 < n_chunks)
            def _():
                start_gather(j + 2, gbuf, gsem)

            pltpu.async_copy(obuf, out_slice(j), ssem)

        def body(i, carry):
            j = 2 * i
            half_step(i, j, gbuf0, obuf0, gsem0, ssem0)
            half_step(i, j + 1, gbuf1, obuf1, gsem1, ssem1)
            return carry

        lax.fori_loop(0, n_chunks // 2, body, 0)
        pltpu.make_async_copy(obuf0, out_slice(n_chunks - 2), ssem0).wait()
        pltpu.make_async_copy(obuf1, out_slice(n_chunks - 1), ssem1).wait()

    return gather_k


_BT = 512


def _transpose_body(x_ref, out_ref):
    out_ref[...] = x_ref[...].T


def _transpose(x2d, B, row_w):
    return pl.pallas_call(
        _transpose_body,
        grid=(B // _BT, row_w // _BT),
        in_specs=[pl.BlockSpec((_BT, _BT), lambda i, j: (i, j))],
        out_specs=pl.BlockSpec((_BT, _BT), lambda i, j: (j, i)),
        out_shape=jax.ShapeDtypeStruct((row_w, B), jnp.float32),
    )(x2d)


def kernel(token_ids, emb_table, W, b):
    B, L = token_ids.shape
    info = plsc.get_sparse_core_info()
    nw = info.num_cores * info.num_subcores

    W2 = jnp.concatenate([W, W], axis=1)
    b2 = jnp.concatenate([b, b]).reshape(1, 2 * NUM_CLASSES)
    proj = _project(emb_table, W2, b2)

    ids2 = (
        token_ids.reshape(B // _BG, _BG, L // _LC, _LC)
        .swapaxes(1, 2)
        .reshape(nw, (B // nw) * L)
        .astype(jnp.int32)
    )
    out2 = _make_gather(B, L, nw)(ids2, proj)
    return out2.T.reshape(L, NUM_CLASSES, B).transpose(2, 0, 1)
